# baseline (speedup 1.0000x reference)
import functools

import jax
import jax.numpy as jnp
from jax import lax
from jax.experimental import pallas as pl
from jax.experimental.pallas import tpu as pltpu
from jax.experimental.pallas import tpu_sc as plsc

LAMBDA = 1.0

_NC = 2
_NS = 16
_NW = _NC * _NS
_L = 16

_ROWS = 16384
_D = 128
_SC_ROWS = 8192
_TC_ROWS = _ROWS - _SC_ROWS
_RPW = _SC_ROWS // _NW
_CR = 128
_CHUNKS = _D // _L
_SCALE = LAMBDA * 0.5 / _ROWS

_TC_BLOCK = 2048


def _make_sc_partials():
    mesh = plsc.VectorSubcoreMesh(core_axis_name="c", subcore_axis_name="s")

    @functools.partial(
        pl.kernel,
        mesh=mesh,
        out_type=jax.ShapeDtypeStruct((_NW, _L), jnp.float32),
        scratch_types=[
            pltpu.VMEM((_CR, _D), jnp.float32),
            pltpu.VMEM((_RPW,), jnp.int32),
            pltpu.VMEM((_RPW, _L), jnp.float32),
            pltpu.VMEM((1, _D), jnp.float32),
            pltpu.VMEM((1, _D), jnp.float32),
            pltpu.VMEM((_L,), jnp.float32),
        ],
    )
    def sc_partials(feat_hbm, lab_hbm, c0_hbm, c1_hbm, out_hbm,
                    feat_v, lab_v, lab16_v, c0_v, c1_v, acc_v):
        wid = lax.axis_index("s") * _NC + lax.axis_index("c")
        base = wid * _RPW
        pltpu.sync_copy(lab_hbm.at[pl.ds(base, _RPW)], lab_v)
        pltpu.sync_copy(c0_hbm, c0_v)
        pltpu.sync_copy(c1_hbm, c1_v)

        c0 = [c0_v[0, pl.ds(j * _L, _L)] for j in range(_CHUNKS)]
        dlt = [c1_v[0, pl.ds(j * _L, _L)] - c0[j] for j in range(_CHUNKS)]

        def expand_body(g, _):
            lvf = lab_v[pl.ds(g * _L, _L)].astype(jnp.float32)
            for k in range(_L):
                lab16_v[g * _L + k, :] = jnp.full((_L,), lvf[k],
                                                  jnp.float32)
            return 0

        lax.fori_loop(0, _RPW // _L, expand_body, 0)

        def chunk_body(ci, acc):
            pltpu.sync_copy(feat_hbm.at[pl.ds(base + ci * _CR, _CR)],
                            feat_v)

            def row_body(r, acc):
                lf = lab16_v[ci * _CR + r, :]
                for j in range(_CHUNKS):
                    t = (feat_v[r, pl.ds(j * _L, _L)]
                         - c0[j] - lf * dlt[j])
                    acc = acc + t * t
                return acc

            return lax.fori_loop(0, _CR, row_body, acc, unroll=2)

        acc = lax.fori_loop(0, _RPW // _CR, chunk_body,
                            jnp.zeros((_L,), jnp.float32))
        acc_v[...] = acc * _SCALE
        pltpu.sync_copy(acc_v, out_hbm.at[wid])

    return sc_partials


_sc_partials = _make_sc_partials()


def _tc_body(feat_ref, lab_ref, c0_ref, c1_ref, out_ref):
    i = pl.program_id(0)
    lab = lab_ref[...]
    sel = jnp.where(lab == 0, c0_ref[...], c1_ref[...])
    d = feat_ref[...] - sel
    s = jnp.sum(d * d) * _SCALE

    @pl.when(i == 0)
    def _():
        out_ref[0, 0] = 0.0

    out_ref[0, 0] += s


def _tc_partial(feat, lab3, proto_0, proto_1):
    nb = _TC_ROWS // _TC_BLOCK
    return pl.pallas_call(
        _tc_body,
        grid=(nb,),
        in_specs=[
            pl.BlockSpec((_TC_BLOCK, _D), lambda i: (i, 0)),
            pl.BlockSpec((_TC_BLOCK, 1), lambda i: (i, 0)),
            pl.BlockSpec((1, _D), lambda i: (0, 0)),
            pl.BlockSpec((1, _D), lambda i: (0, 0)),
        ],
        out_specs=pl.BlockSpec(
            block_shape=(1, 1), index_map=lambda i: (0, 0),
            memory_space=pltpu.SMEM),
        out_shape=jax.ShapeDtypeStruct((1, 1), jnp.float32),
    )(feat, lab3, proto_0, proto_1)


def kernel(features, labels, proto_0, proto_1):
    labels = labels.astype(jnp.int32)
    sc_part = _sc_partials(features[:_SC_ROWS], labels[:_SC_ROWS],
                           proto_0, proto_1)
    lab2 = labels[_SC_ROWS:].reshape(_TC_ROWS, 1)
    tc_part = _tc_partial(features[_SC_ROWS:], lab2, proto_0, proto_1)
    return jnp.sum(sc_part) + tc_part[0, 0]


# speedup vs baseline: 1.1649x; 1.1649x over previous
import functools

import jax
import jax.numpy as jnp
from jax import lax
from jax.experimental import pallas as pl
from jax.experimental.pallas import tpu as pltpu
from jax.experimental.pallas import tpu_sc as plsc

LAMBDA = 1.0

_NC = 2
_NS = 16
_NW = _NC * _NS
_L = 16

_ROWS = 16384
_D = 128
_SC_ROWS = 6144
_TC_ROWS = _ROWS - _SC_ROWS
_RPW = _SC_ROWS // _NW
_CHUNKS = _D // _L
_SCALE = LAMBDA * 0.5 / _ROWS

_TC_BLOCK = 2048
_TC_OFF = _SC_ROWS // _TC_BLOCK


def _make_sc_partials():
    mesh = plsc.VectorSubcoreMesh(core_axis_name="c", subcore_axis_name="s")

    @functools.partial(
        pl.kernel,
        mesh=mesh,
        out_type=jax.ShapeDtypeStruct((_NW, _L), jnp.float32),
        scratch_types=[
            pltpu.VMEM((_RPW, _D), jnp.float32),
            pltpu.VMEM((_RPW,), jnp.int32),
            pltpu.VMEM((_RPW, _L), jnp.float32),
            pltpu.VMEM((1, _D), jnp.float32),
            pltpu.VMEM((1, _D), jnp.float32),
            pltpu.VMEM((_L,), jnp.float32),
        ],
    )
    def sc_partials(feat_hbm, lab_hbm, c0_hbm, c1_hbm, out_hbm,
                    feat_v, lab_v, lab16_v, c0_v, c1_v, acc_v):
        wid = lax.axis_index("s") * _NC + lax.axis_index("c")
        base = wid * _RPW
        pltpu.sync_copy(feat_hbm.at[pl.ds(base, _RPW)], feat_v)
        pltpu.sync_copy(lab_hbm.at[pl.ds(base, _RPW)], lab_v)
        pltpu.sync_copy(c0_hbm, c0_v)
        pltpu.sync_copy(c1_hbm, c1_v)

        c0 = [c0_v[0, pl.ds(j * _L, _L)] for j in range(_CHUNKS)]
        dlt = [c1_v[0, pl.ds(j * _L, _L)] - c0[j] for j in range(_CHUNKS)]

        def expand_body(g, _):
            lvf = lab_v[pl.ds(g * _L, _L)].astype(jnp.float32)
            for k in range(_L):
                lab16_v[g * _L + k, :] = jnp.full((_L,), lvf[k],
                                                  jnp.float32)
            return 0

        lax.fori_loop(0, _RPW // _L, expand_body, 0)

        def row_body(r, acc):
            lf = lab16_v[r, :]
            for j in range(_CHUNKS):
                t = feat_v[r, pl.ds(j * _L, _L)] - c0[j] - lf * dlt[j]
                acc = acc + t * t
            return acc

        acc = lax.fori_loop(0, _RPW, row_body,
                            jnp.zeros((_L,), jnp.float32), unroll=2)
        acc_v[...] = acc * _SCALE
        pltpu.sync_copy(acc_v, out_hbm.at[wid])

    return sc_partials


_sc_partials = _make_sc_partials()


def _tc_body(feat_ref, lab_ref, c0_ref, c1_ref, out_ref):
    i = pl.program_id(0)
    lab = lab_ref[...]
    sel = jnp.where(lab == 0, c0_ref[...], c1_ref[...])
    d = feat_ref[...] - sel
    s = jnp.sum(d * d) * _SCALE

    @pl.when(i == 0)
    def _():
        out_ref[0, 0] = 0.0

    out_ref[0, 0] += s


def _tc_partial(feat, lab2, proto_0, proto_1):
    nb = _TC_ROWS // _TC_BLOCK
    return pl.pallas_call(
        _tc_body,
        grid=(nb,),
        in_specs=[
            pl.BlockSpec((_TC_BLOCK, _D), lambda i: (i + _TC_OFF, 0)),
            pl.BlockSpec((_TC_BLOCK, 1), lambda i: (i + _TC_OFF, 0)),
            pl.BlockSpec((1, _D), lambda i: (0, 0)),
            pl.BlockSpec((1, _D), lambda i: (0, 0)),
        ],
        out_specs=pl.BlockSpec(
            block_shape=(1, 1), index_map=lambda i: (0, 0),
            memory_space=pltpu.SMEM),
        out_shape=jax.ShapeDtypeStruct((1, 1), jnp.float32),
    )(feat, lab2, proto_0, proto_1)


def kernel(features, labels, proto_0, proto_1):
    labels = labels.astype(jnp.int32)
    sc_part = _sc_partials(features, labels, proto_0, proto_1)
    tc_part = _tc_partial(features, labels.reshape(_ROWS, 1),
                          proto_0, proto_1)
    return jnp.sum(sc_part) + tc_part[0, 0]
